# baseline (speedup 1.0000x reference)
import functools

import jax
import jax.numpy as jnp
from jax import lax
from jax.experimental import pallas as pl
from jax.experimental.pallas import tpu as pltpu
from jax.experimental.pallas import tpu_sc as plsc

_VOCAB = 1000
_VPAD = 1024
_HIDDEN = 2048
_BATCH = 4
_SEQ = 4096
_TOKENS = _BATCH * _SEQ
_NUM_WORKERS = 32
_TROWS = _VOCAB // 8
_NTASK = _BATCH * _TROWS
_TASK_PER_W = 16
_NCT = _SEQ // 128


def _table_body(w_ref, emb_ref, b_ref, out_ref):
    out_ref[...] = jnp.pad(
        lax.dot_general(
            w_ref[...], emb_ref[...],
            dimension_numbers=(((1,), (1,)), ((), ())),
            preferred_element_type=jnp.float32,
        ) + b_ref[...],
        ((0, 0), (0, _VPAD - _VOCAB)),
    )


def _build_table(embed_table, lm_w, lm_bcol):
    return pl.pallas_call(
        _table_body,
        out_shape=jax.ShapeDtypeStruct((_VOCAB, _VPAD), jnp.float32),
    )(lm_w, embed_table, lm_bcol)


_sc_mesh = plsc.VectorSubcoreMesh(core_axis_name="c", subcore_axis_name="s")


@functools.partial(
    pl.kernel,
    mesh=_sc_mesh,
    out_type=jax.ShapeDtypeStruct((_BATCH, _VOCAB, _SEQ), jnp.float32),
    scratch_types=[
        pltpu.VMEM((_SEQ,), jnp.int32),
        pltpu.VMEM((8, 8, 128), jnp.float32),
        pltpu.VMEM((_NCT, 8, 128), jnp.float32),
    ],
    compiler_params=pltpu.CompilerParams(needs_layout_passes=False),
)
def _select_sc(tab_hbm, ids_hbm, out_hbm, ids_v, rows_v, strip_v):
    wid = lax.axis_index("s") * 2 + lax.axis_index("c")

    for i in range(_TASK_PER_W):
        task = wid + i * _NUM_WORKERS

        @pl.when(task < _NTASK)
        def _():
            b = task // _TROWS
            tr = task % _TROWS
            pltpu.sync_copy(ids_hbm.at[pl.ds(b * _SEQ, _SEQ)], ids_v)
            pltpu.sync_copy(tab_hbm.at[pl.ds(tr * 8, 8)], rows_v)

            @plsc.parallel_loop(0, _SEQ // 16, unroll=2)
            def body(sv):
                idx = plsc.load_gather(
                    ids_v, [sv * 16 + lax.iota(jnp.int32, 16)])
                u_hi = lax.shift_right_logical(idx, 7)
                u_lo = lax.bitwise_and(idx, 127)
                ct_vec = jnp.full((16,), sv // 8, jnp.int32)
                k_vec = (sv % 8) * 16 + lax.iota(jnp.int32, 16)
                for w in range(8):
                    w_vec = jnp.full((16,), w, jnp.int32)
                    vals = plsc.load_gather(rows_v, [w_vec, u_hi, u_lo])
                    plsc.store_scatter(strip_v, [ct_vec, w_vec, k_vec], vals)

            for ct in range(_NCT):
                pltpu.sync_copy(
                    strip_v.at[ct],
                    out_hbm.at[b, pl.ds(tr * 8, 8), pl.ds(ct * 128, 128)])


def kernel(input_ids, embed_table, gates, lm_w, lm_b):
    del gates
    tab = _build_table(embed_table, lm_w, lm_b.reshape(_VOCAB, 1))
    tab = tab.reshape(_VOCAB, 8, 128)
    ids = input_ids.reshape(_TOKENS).astype(jnp.int32)
    out_t = _select_sc(tab, ids)
    return out_t.transpose(0, 2, 1)
